# baseline (speedup 1.0000x reference)
import jax
import jax.numpy as jnp
from jax import lax
from jax.experimental import pallas as pl
from jax.experimental.pallas import tpu as pltpu
from jax.experimental.pallas import tpu_sc as plsc

N = 10000
E = 80000
C = 16
TM = 16
NS = 16
PW = 128
EPAD = 81920
PIECES = EPAD // PW
PPT = PIECES // NS


def _sc_degree_body(eh_i, eh_w, ev_i, ev_w, zeros_hbm, out,
                    idx_v, w_v, acc, sem):
    core = lax.axis_index("c")
    sub = lax.axis_index("s")

    @pl.when(sub == 0)
    def _zero():
        pltpu.sync_copy(zeros_hbm, acc)

    plsc.subcore_barrier()
    base = sub * PPT

    def _accumulate(idx3, w2):
        pltpu.sync_copy(idx3.at[0, pl.ds(base, PPT)], idx_v.at[0])
        pltpu.sync_copy(idx3.at[1, pl.ds(base, PPT)], idx_v.at[1])
        pltpu.sync_copy(w2.at[pl.ds(base, PPT)], w_v)

        descs = []
        for j in range(PPT):
            for r in (0, 1):
                descs.append(pltpu.async_copy(
                    w_v.at[j], acc.at[idx_v.at[r, j]], sem, add=True))
        for d in descs:
            d.wait()

    @pl.when(core == 0)
    def _h():
        _accumulate(eh_i, eh_w)

    @pl.when(core == 1)
    def _v():
        _accumulate(ev_i, ev_w)

    plsc.subcore_barrier()

    @pl.when(sub == 0)
    def _out():
        pltpu.sync_copy(acc, out.at[core])


def _sc_degree(eh_idx, eh_w, ev_idx, ev_w):
    mesh = plsc.VectorSubcoreMesh(core_axis_name="c", subcore_axis_name="s")
    zeros = jnp.zeros((N,), jnp.float32)
    pad = EPAD - E
    pad_idx = jnp.broadcast_to((jnp.arange(pad, dtype=jnp.int32) * 53) % N,
                               (2, pad))

    def prep(idx, w):
        idx_p = jnp.concatenate([idx, pad_idx], axis=1)
        w_p = jnp.pad(w, (0, pad))
        return idx_p.reshape(2, PIECES, PW), w_p.reshape(PIECES, PW)

    ehi, ehw = prep(eh_idx, eh_w)
    evi, evw = prep(ev_idx, ev_w)
    k = pl.kernel(
        _sc_degree_body,
        out_type=jax.ShapeDtypeStruct((2, N), jnp.float32),
        mesh=mesh,
        scratch_types=[
            pltpu.VMEM((2, PPT, PW), jnp.int32),
            pltpu.VMEM((PPT, PW), jnp.float32),
            pltpu.VMEM_SHARED((N,), jnp.float32),
            pltpu.SemaphoreType.DMA,
        ],
    )
    return k(ehi, ehw, evi, evw, zeros)


def _ln_relu(o, lnw, lnb, ones_row):
    o = jnp.maximum(o, 0.0)
    mu = jnp.dot(ones_row, o, preferred_element_type=jnp.float32,
                 precision=lax.Precision.HIGHEST) * (1.0 / C)
    d = o - mu
    var = jnp.dot(ones_row, d * d, preferred_element_type=jnp.float32,
                  precision=lax.Precision.HIGHEST) * (1.0 / C)
    return d * lax.rsqrt(var + 1e-5) * lnw + lnb


def _tc_dense_body(x_ref, sw_ref, p_ref, out_ref):
    f32 = jnp.float32
    wht = p_ref[:, 0:C]
    wvt = p_ref[:, C:2 * C]
    bias = p_ref[:, 2 * C:2 * C + 1]
    lnw = p_ref[:, 2 * C + 1:2 * C + 2]
    lnb = p_ref[:, 2 * C + 2:2 * C + 3]
    wst = jnp.concatenate([wht, wvt], axis=0)
    sw_h = sw_ref[0:1, :]
    sw_v = sw_ref[1:2, :]
    deg_h = 1.0 + sw_h
    deg_v = 1.0 + sw_v
    a_h = sw_h * lax.rsqrt(deg_h)
    a_v = sw_v * lax.rsqrt(deg_v)
    c_h = 1.0 / deg_h
    c_v = 1.0 / deg_v
    ones_row = jnp.ones((1, C), f32)
    for t in range(8):
        p0 = jnp.dot(wst, x_ref[0, :, t, :], preferred_element_type=f32)
        p1 = jnp.dot(wst, x_ref[1, :, t, :], preferred_element_type=f32)
        o0 = p0[0:C, :] + p0[C:2 * C, :] + bias
        o1 = (a_h * p0[0:C, :] + a_v * p0[C:2 * C, :]
              + c_h * p1[0:C, :] + c_v * p1[C:2 * C, :] + bias)
        out_ref[0, :, t, :] = _ln_relu(o0, lnw, lnb, ones_row)
        out_ref[1, :, t, :] = _ln_relu(o1, lnw, lnb, ones_row)


def _tc_dense(x_room, sumw, W_h, b_h, W_v, b_v, ln_weight, ln_bias):
    params = jnp.concatenate(
        [W_h.T, W_v.T, (b_h + b_v)[:, None], ln_weight[:, None],
         ln_bias[:, None]], axis=1)
    return pl.pallas_call(
        _tc_dense_body,
        grid=(2,),
        in_specs=[pl.BlockSpec((2, C, 8, N), lambda i: (0, 0, i, 0)),
                  pl.BlockSpec((2, N), lambda i: (0, 0)),
                  pl.BlockSpec((C, 2 * C + 3), lambda i: (0, 0))],
        out_specs=pl.BlockSpec((2, C, 8, N), lambda i: (0, 0, i, 0)),
        out_shape=jax.ShapeDtypeStruct((2, C, TM, N), jnp.float32),
    )(x_room, sumw, params)


def kernel(x_room, edge_h_index, edge_h_weight, edge_v_index, edge_v_weight,
           W_h, b_h, W_v, b_v, ln_weight, ln_bias):
    sumw = _sc_degree(edge_h_index, edge_h_weight,
                      edge_v_index, edge_v_weight)
    return _tc_dense(x_room, sumw, W_h, b_h, W_v, b_v, ln_weight, ln_bias)


# speedup vs baseline: 1.3684x; 1.3684x over previous
import jax
import jax.numpy as jnp
from jax import lax
from jax.experimental import pallas as pl
from jax.experimental.pallas import tpu as pltpu
from jax.experimental.pallas import tpu_sc as plsc

N = 10000
E = 80000
C = 16
TM = 16
NS = 16
PW = 128
EPAD = 81920
PIECES = EPAD // PW
PPT = PIECES // NS


def _sc_degree_body(eh_i, eh_w, ev_i, ev_w, zeros_hbm, out,
                    idx_v, w_v, acc, sem):
    core = lax.axis_index("c")
    sub = lax.axis_index("s")

    @pl.when(sub == 0)
    def _zero():
        pltpu.sync_copy(zeros_hbm, acc)

    plsc.subcore_barrier()
    base = sub * PPT

    def _accumulate(idx3, w2):
        pltpu.sync_copy(idx3.at[0, pl.ds(base, PPT)], idx_v.at[0])
        pltpu.sync_copy(idx3.at[1, pl.ds(base, PPT)], idx_v.at[1])
        pltpu.sync_copy(w2.at[pl.ds(base, PPT)], w_v)

        descs = []
        for j in range(PPT):
            for r in (0, 1):
                descs.append(pltpu.async_copy(
                    w_v.at[j], acc.at[idx_v.at[r, j]], sem, add=True))
        for d in descs:
            d.wait()

    @pl.when(core == 0)
    def _h():
        _accumulate(eh_i, eh_w)

    @pl.when(core == 1)
    def _v():
        _accumulate(ev_i, ev_w)

    plsc.subcore_barrier()

    @pl.when(sub == 0)
    def _out():
        pltpu.sync_copy(acc, out.at[core])


def _sc_degree(eh_idx, eh_w, ev_idx, ev_w):
    mesh = plsc.VectorSubcoreMesh(core_axis_name="c", subcore_axis_name="s")
    zeros = jnp.zeros((N,), jnp.float32)
    pad = EPAD - E
    pad_idx = jnp.broadcast_to((jnp.arange(pad, dtype=jnp.int32) * 53) % N,
                               (2, pad))

    def prep(idx, w):
        idx_p = jnp.concatenate([idx, pad_idx], axis=1)
        w_p = jnp.pad(w, (0, pad))
        return idx_p.reshape(2, PIECES, PW), w_p.reshape(PIECES, PW)

    ehi, ehw = prep(eh_idx, eh_w)
    evi, evw = prep(ev_idx, ev_w)
    k = pl.kernel(
        _sc_degree_body,
        out_type=jax.ShapeDtypeStruct((2, N), jnp.float32),
        mesh=mesh,
        scratch_types=[
            pltpu.VMEM((2, PPT, PW), jnp.int32),
            pltpu.VMEM((PPT, PW), jnp.float32),
            pltpu.VMEM_SHARED((N,), jnp.float32),
            pltpu.SemaphoreType.DMA,
        ],
    )
    return k(ehi, ehw, evi, evw, zeros)


def _ln_relu(o, lnw, lnb):
    o = jnp.maximum(o, 0.0)
    mu = jnp.mean(o, axis=0, keepdims=True)
    d = o - mu
    var = jnp.mean(d * d, axis=0, keepdims=True)
    return d * lax.rsqrt(var + 1e-5) * lnw + lnb


def _tc_dense_body(x_ref, sw_ref, p_ref, out_ref):
    f32 = jnp.float32
    wht = p_ref[:, 0:C]
    wvt = p_ref[:, C:2 * C]
    bias = p_ref[:, 2 * C:2 * C + 1]
    lnw = p_ref[:, 2 * C + 1:2 * C + 2]
    lnb = p_ref[:, 2 * C + 2:2 * C + 3]
    wst = jnp.concatenate([wht, wvt], axis=0)
    sw_h = sw_ref[0:1, :]
    sw_v = sw_ref[1:2, :]
    deg_h = 1.0 + sw_h
    deg_v = 1.0 + sw_v
    a_h = sw_h * lax.rsqrt(deg_h)
    a_v = sw_v * lax.rsqrt(deg_v)
    c_h = 1.0 / deg_h
    c_v = 1.0 / deg_v
    for t in range(8):
        p0 = jnp.dot(wst, x_ref[0, :, t, :], preferred_element_type=f32)
        p1 = jnp.dot(wst, x_ref[1, :, t, :], preferred_element_type=f32)
        o0 = p0[0:C, :] + p0[C:2 * C, :] + bias
        o1 = (a_h * p0[0:C, :] + a_v * p0[C:2 * C, :]
              + c_h * p1[0:C, :] + c_v * p1[C:2 * C, :] + bias)
        out_ref[0, :, t, :] = _ln_relu(o0, lnw, lnb)
        out_ref[1, :, t, :] = _ln_relu(o1, lnw, lnb)


def _tc_dense(x_room, sumw, W_h, b_h, W_v, b_v, ln_weight, ln_bias):
    params = jnp.concatenate(
        [W_h.T, W_v.T, (b_h + b_v)[:, None], ln_weight[:, None],
         ln_bias[:, None]], axis=1)
    return pl.pallas_call(
        _tc_dense_body,
        grid=(2,),
        in_specs=[pl.BlockSpec((2, C, 8, N), lambda i: (0, 0, i, 0)),
                  pl.BlockSpec((2, N), lambda i: (0, 0)),
                  pl.BlockSpec((C, 2 * C + 3), lambda i: (0, 0))],
        out_specs=pl.BlockSpec((2, C, 8, N), lambda i: (0, 0, i, 0)),
        out_shape=jax.ShapeDtypeStruct((2, C, TM, N), jnp.float32),
    )(x_room, sumw, params)


def kernel(x_room, edge_h_index, edge_h_weight, edge_v_index, edge_v_weight,
           W_h, b_h, W_v, b_v, ln_weight, ln_bias):
    sumw = _sc_degree(edge_h_index, edge_h_weight,
                      edge_v_index, edge_v_weight)
    return _tc_dense(x_room, sumw, W_h, b_h, W_v, b_v, ln_weight, ln_bias)
